# Initial kernel scaffold; baseline (speedup 1.0000x reference)
import jax
import jax.numpy as jnp
from jax.experimental import pallas as pl


def kernel(x, Wr, w1s, w3s, w2s, W1, W2):
    raise NotImplementedError("write your pallas kernel here")


# speedup vs baseline: 1.4499x; 1.4499x over previous
import functools

import jax
import jax.numpy as jnp
from jax import lax
from jax.experimental import pallas as pl
from jax.experimental.pallas import tpu as pltpu
from jax.experimental.pallas import tpu_sc as plsc

TILE = 256


def _router_body(x_ref, wr_ref, idx_ref, g_ref):
    x = x_ref[...]
    wr = wr_ref[...]
    logits = lax.dot_general(x, wr, (((1,), (1,)), ((), ())),
                             preferred_element_type=jnp.float32)
    s = jax.nn.sigmoid(logits)
    n, e = s.shape
    col = lax.broadcasted_iota(jnp.int32, (n, e), 1)
    m1 = jnp.max(s, axis=1, keepdims=True)
    i1 = jnp.min(jnp.where(s == m1, col, e), axis=1, keepdims=True)
    s2 = jnp.where(col == i1, -jnp.inf, s)
    m2 = jnp.max(s2, axis=1, keepdims=True)
    i2 = jnp.min(jnp.where(s2 == m2, col, e), axis=1, keepdims=True)
    denom = m1 + m2
    safe = denom > 1e-9
    g1 = jnp.where(safe, m1 / (denom + 1e-9), 0.5)
    g2 = jnp.where(safe, m2 / (denom + 1e-9), 0.5)
    idx_ref[...] = jnp.concatenate([i1, i2], axis=1)
    g_ref[...] = jnp.concatenate([g1, g2], axis=1)


def _router(xf, Wr):
    n = xf.shape[0]
    return pl.pallas_call(
        _router_body,
        out_shape=(jax.ShapeDtypeStruct((n, 2), jnp.int32),
                   jax.ShapeDtypeStruct((n, 2), jnp.float32)),
    )(xf, Wr)


def _shared_body(x_ref, w1_ref, w3_ref, w2_ref, o_ref):
    x = x_ref[...]
    a = lax.dot_general(x, w1_ref[...], (((1,), (1,)), ((), ())),
                        preferred_element_type=jnp.float32)
    b = lax.dot_general(x, w3_ref[...], (((1,), (1,)), ((), ())),
                        preferred_element_type=jnp.float32)
    h = a * jax.nn.sigmoid(a) * b
    o_ref[...] = lax.dot_general(h, w2_ref[...], (((1,), (1,)), ((), ())),
                                 preferred_element_type=jnp.float32)


def _shared(xf, w1s, w3s, w2s):
    n, d = xf.shape
    hs = w1s.shape[0]
    bt = 256
    return pl.pallas_call(
        _shared_body,
        grid=(n // bt,),
        in_specs=[
            pl.BlockSpec((bt, d), lambda i: (i, 0)),
            pl.BlockSpec((hs, d), lambda i: (0, 0)),
            pl.BlockSpec((hs, d), lambda i: (0, 0)),
            pl.BlockSpec((d, hs), lambda i: (0, 0)),
        ],
        out_specs=pl.BlockSpec((bt, d), lambda i: (i, 0)),
        out_shape=jax.ShapeDtypeStruct((n, d), jnp.float32),
    )(xf, w1s, w3s, w2s)


def _ffn_body(te_ref, xs_ref, w1_ref, w2_ref, gs_ref, eo_ref):
    del te_ref
    xb = xs_ref[...]
    h = lax.dot_general(xb, w1_ref[0], (((1,), (1,)), ((), ())),
                        preferred_element_type=jnp.float32)
    h = 0.5 * h * (1.0 + lax.erf(h * 0.7071067811865476))
    eo = lax.dot_general(h, w2_ref[0], (((1,), (1,)), ((), ())),
                         preferred_element_type=jnp.float32)
    eo_ref[...] = eo * gs_ref[...]


def _ffn(te, xs, W1, W2, gs2d):
    np_, d = xs.shape
    _, hr, _ = W1.shape
    nt = np_ // TILE
    grid_spec = pltpu.PrefetchScalarGridSpec(
        num_scalar_prefetch=1,
        grid=(nt,),
        in_specs=[
            pl.BlockSpec((TILE, d), lambda t, te_r: (t, 0)),
            pl.BlockSpec((1, hr, d), lambda t, te_r: (te_r[t], 0, 0)),
            pl.BlockSpec((1, d, hr), lambda t, te_r: (te_r[t], 0, 0)),
            pl.BlockSpec((TILE, 1), lambda t, te_r: (t, 0)),
        ],
        out_specs=pl.BlockSpec((TILE, d), lambda t, te_r: (t, 0)),
    )
    return pl.pallas_call(
        _ffn_body,
        grid_spec=grid_spec,
        out_shape=jax.ShapeDtypeStruct((np_, d), jnp.float32),
    )(te, xs, W1, W2, gs2d)


def _sc_gather(xf, st, np_):
    d = xf.shape[1]
    info = plsc.get_sparse_core_info()
    nw = info.num_cores * info.num_subcores
    rows_per = np_ // nw
    ch = 48 if rows_per % 48 == 0 else rows_per
    n_ch = rows_per // ch
    mesh = plsc.VectorSubcoreMesh(core_axis_name="c", subcore_axis_name="s")

    @functools.partial(
        pl.kernel, mesh=mesh,
        out_type=jax.ShapeDtypeStruct((np_, d), jnp.float32),
        scratch_types=[pltpu.VMEM((ch,), jnp.int32),
                       pltpu.VMEM((ch, d), jnp.float32),
                       pltpu.SemaphoreType.DMA],
    )
    def k(x_hbm, st_hbm, out_hbm, idx_v, rows_v, sem):
        wid = lax.axis_index("s") * info.num_cores + lax.axis_index("c")
        base0 = wid * rows_per
        for c in range(n_ch):
            base = base0 + c * ch
            pltpu.sync_copy(st_hbm.at[pl.ds(base, ch)], idx_v)
            pltpu.async_copy(x_hbm.at[idx_v], rows_v, sem).wait()
            pltpu.sync_copy(rows_v, out_hbm.at[pl.ds(base, ch)])

    return k(xf, st)


def _sc_combine(shared, eo, p0, p1):
    n, d = shared.shape
    info = plsc.get_sparse_core_info()
    nw = info.num_cores * info.num_subcores
    tok_per = n // nw
    ch = 16
    n_ch = tok_per // ch
    mesh = plsc.VectorSubcoreMesh(core_axis_name="c", subcore_axis_name="s")

    @functools.partial(
        pl.kernel, mesh=mesh,
        out_type=jax.ShapeDtypeStruct((n, d), jnp.float32),
        scratch_types=[pltpu.VMEM((ch,), jnp.int32),
                       pltpu.VMEM((ch,), jnp.int32),
                       pltpu.VMEM((ch, d), jnp.float32),
                       pltpu.VMEM((ch, d), jnp.float32),
                       pltpu.VMEM((ch, d), jnp.float32),
                       pltpu.SemaphoreType.DMA],
    )
    def k(sh_hbm, eo_hbm, p0_hbm, p1_hbm, out_hbm, i0_v, i1_v, sh_v, a_v, b_v,
          sem):
        wid = lax.axis_index("s") * info.num_cores + lax.axis_index("c")
        base0 = wid * tok_per
        for c in range(n_ch):
            base = base0 + c * ch
            pltpu.sync_copy(p0_hbm.at[pl.ds(base, ch)], i0_v)
            pltpu.sync_copy(p1_hbm.at[pl.ds(base, ch)], i1_v)
            pltpu.sync_copy(sh_hbm.at[pl.ds(base, ch)], sh_v)
            cp_a = pltpu.async_copy(eo_hbm.at[i0_v], a_v, sem)
            cp_b = pltpu.async_copy(eo_hbm.at[i1_v], b_v, sem)
            cp_a.wait()
            cp_b.wait()

            def row(i, carry):
                def chunk(j, carry2):
                    sl = pl.ds(j * 16, 16)
                    sh_v[i, sl] = sh_v[i, sl] + a_v[i, sl] + b_v[i, sl]
                    return carry2
                return lax.fori_loop(0, d // 16, chunk, carry)

            lax.fori_loop(0, ch, row, 0)
            pltpu.sync_copy(sh_v, out_hbm.at[pl.ds(base, ch)])

    return k(shared, eo, p0, p1)


def _slot_layout(idx2, g2, n, e):
    nt = (n * 2) // TILE + e
    np_ = nt * TILE
    sel = (jax.nn.one_hot(idx2[:, 0], e, dtype=jnp.int32)
           + jax.nn.one_hot(idx2[:, 1], e, dtype=jnp.int32))
    cnt = jnp.sum(sel, axis=0)
    rank = jnp.cumsum(sel, axis=0) - sel
    gpad = ((cnt + TILE - 1) // TILE) * TILE
    ends = jnp.cumsum(gpad)
    off = ends - gpad
    pos_ne = off[None, :] + rank
    pos2 = jnp.take_along_axis(pos_ne, idx2, axis=1)
    tok = jnp.arange(n, dtype=jnp.int32)
    st = (jnp.zeros((np_,), jnp.int32)
          .at[pos2[:, 0]].set(tok)
          .at[pos2[:, 1]].set(tok))
    gs = (jnp.zeros((np_,), jnp.float32)
          .at[pos2[:, 0]].set(g2[:, 0])
          .at[pos2[:, 1]].set(g2[:, 1]))
    tile_start = jnp.arange(nt, dtype=jnp.int32) * TILE
    te = jnp.searchsorted(ends, tile_start, side="right").astype(jnp.int32)
    te = jnp.minimum(te, e - 1)
    return st, gs, pos2, te, np_


def kernel(x, Wr, w1s, w3s, w2s, W1, W2):
    bq, tq, d = x.shape
    n = bq * tq
    e = Wr.shape[0]
    xf = x.reshape(n, d)

    idx2, g2 = _router(xf, Wr)
    st, gs, pos2, te, np_ = _slot_layout(idx2, g2, n, e)

    xs = _sc_gather(xf, st, np_)
    shared = _shared(xf, w1s, w3s, w2s)
    eo = _ffn(te, xs, W1, W2, gs[:, None])
    out = _sc_combine(shared, eo, pos2[:, 0], pos2[:, 1])
    return out.reshape(bq, tq, d)
